# Initial kernel scaffold; baseline (speedup 1.0000x reference)
import jax
import jax.numpy as jnp
from jax.experimental import pallas as pl


def kernel(edge_index, edge_attr):
    raise NotImplementedError("write your pallas kernel here")


# speedup vs baseline: 1.0017x; 1.0017x over previous
import jax
import jax.numpy as jnp
from jax.experimental import pallas as pl

_N = 50000


def _mask_fill_body(x_ref, m_ref, o_ref):
    o_ref[...] = jnp.where(m_ref[...] > 0, x_ref[...], 0.0)


def _mask_fill(x2d, m2d):
    rows = x2d.shape[0]
    blk = 640
    assert rows % blk == 0
    return pl.pallas_call(
        _mask_fill_body,
        out_shape=jax.ShapeDtypeStruct(x2d.shape, jnp.float32),
        grid=(rows // blk,),
        in_specs=[
            pl.BlockSpec((blk, 128), lambda i: (i, jnp.int32(0))),
            pl.BlockSpec((blk, 128), lambda i: (i, jnp.int32(0))),
        ],
        out_specs=pl.BlockSpec((blk, 128), lambda i: (i, jnp.int32(0))),
    )(x2d, m2d)


def kernel(edge_index, edge_attr):
    row, col = edge_index[0], edge_index[1]
    full_row = jnp.concatenate([row, col])
    full_col = jnp.concatenate([col, row])
    full_attr = jnp.concatenate([edge_attr, edge_attr], axis=0)
    keys = full_row * _N + full_col
    M = keys.shape[0]
    uniq, inv = jnp.unique(keys, return_inverse=True, size=M, fill_value=-1)
    out = jax.ops.segment_max(full_attr, inv, num_segments=M)
    counts = jax.ops.segment_sum(jnp.ones((M,), jnp.float32), inv, num_segments=M)
    mask = jnp.repeat(counts > 0, 16).astype(jnp.float32)
    D = full_attr.shape[1]
    x2d = out.reshape(M * D // 128, 128)
    m2d = mask.reshape(M * D // 128, 128)
    out = _mask_fill(x2d, m2d).reshape(M, D)
    new_edge_index = jnp.stack([uniq // _N, uniq % _N])
    return new_edge_index, out


# speedup vs baseline: 26.2900x; 26.2460x over previous
import functools

import jax
import jax.numpy as jnp
from jax import lax
from jax.experimental import pallas as pl
from jax.experimental.pallas import tpu as pltpu
from jax.experimental.pallas import tpu_sc as plsc

_N = 50000
_NC, _NS = 2, 16
_NW = _NC * _NS
_BITS = (11, 11, 10)
_SHIFTS = (0, 11, 22)
_BK = 2000

_MESH = plsc.VectorSubcoreMesh(core_axis_name="c", subcore_axis_name="s")
_CP = pltpu.CompilerParams(
    needs_layout_passes=False, use_tc_tiling_on_sc=False)

_i32 = jnp.int32
_f32 = jnp.float32


def _wid():
    return lax.axis_index("s") * _NC + lax.axis_index("c")


def _lane():
    return lax.iota(_i32, 16)


def _srl(x, s):
    return lax.shift_right_logical(x, jnp.full((16,), s, _i32))


def _sll(x, s):
    return lax.shift_left(x, jnp.full((16,), s, _i32))


def _splat(x):
    return jnp.zeros((16,), _i32) + x


def _fori(n, body, init):
    return lax.fori_loop(_i32(0), _i32(n), body, init)


def _al(x):
    return pl.multiple_of(x, 8)


def _make_kernels(E):
    M = 2 * E
    CH = M // _NW
    assert CH % _BK == 0 and E % CH == 0
    NB = CH // _BK
    NV = _BK // 16

    @functools.partial(
        pl.kernel,
        out_type=jax.ShapeDtypeStruct((M,), _i32),
        mesh=_MESH, compiler_params=_CP,
        scratch_types=[pltpu.VMEM((_BK,), _i32),
                       pltpu.VMEM((_BK,), _i32),
                       pltpu.VMEM((_BK,), _i32)])
    def k_build(row_h, col_h, key_h, rbuf, cbuf, kbuf):
        base = _wid() * CH

        def bt(t, _):
            b = base + t * _BK
            flip = b >= E
            eb = jnp.where(flip, b - E, b)
            pltpu.sync_copy(row_h.at[pl.ds(_al(eb), _BK)], rbuf)
            pltpu.sync_copy(col_h.at[pl.ds(_al(eb), _BK)], cbuf)

            def bi(i, _2):
                r = rbuf[pl.ds(i * 16, 16)]
                c = cbuf[pl.ds(i * 16, 16)]
                fwd = _sll(r, 16) | c
                rev = _sll(c, 16) | r
                kbuf[pl.ds(i * 16, 16)] = jnp.where(flip, rev, fwd)
                return 0

            _fori(NV, bi, 0)
            pltpu.sync_copy(kbuf, key_h.at[pl.ds(_al(b), _BK)])
            return 0

        _fori(NB, bt, 0)

    def make_hist(shift, bins):
        @functools.partial(
            pl.kernel,
            out_type=jax.ShapeDtypeStruct((_NW * bins,), _i32),
            mesh=_MESH, compiler_params=_CP,
            scratch_types=[pltpu.VMEM((_BK,), _i32),
                           pltpu.VMEM((bins,), _i32)])
        def k_hist(key_h, hist_h, kbuf, htbl):
            w = _wid()
            base = w * CH

            def z(j, _):
                htbl[pl.ds(j * 16, 16)] = jnp.zeros((16,), _i32)
                return 0

            _fori(bins // 16, z, 0)

            def bt(t, _):
                pltpu.sync_copy(key_h.at[pl.ds(_al(base + t * _BK), _BK)], kbuf)

                def bi(i, _2):
                    kk = kbuf[pl.ds(i * 16, 16)]
                    d = _srl(kk, shift) & _i32(bins - 1)
                    rank, lastm = plsc.scan_count(d)
                    plsc.addupdate_scatter(htbl, [d], rank, mask=lastm)
                    return 0

                _fori(NV, bi, 0)
                return 0

            _fori(NB, bt, 0)
            pltpu.sync_copy(htbl, hist_h.at[pl.ds(_al(w * bins), bins)])

        return k_hist

    def make_scat(shift, bins, first):
        ins = 2 if first else 3

        def body(*refs):
            if first:
                key_h, hist_h = refs[0], refs[1]
                id_h = None
                keyo_h, ido_h = refs[2], refs[3]
                htbl, offtbl, kbuf, ibuf, pbuf, sem = refs[4:]
            else:
                key_h, id_h, hist_h = refs[0], refs[1], refs[2]
                keyo_h, ido_h = refs[3], refs[4]
                htbl, offtbl, kbuf, ibuf, pbuf, sem = refs[5:]
            w = _wid()
            base = w * CH
            pltpu.sync_copy(hist_h, htbl)

            def ob(jb, carry):
                tot = jnp.zeros((16,), _i32)
                mine = jnp.zeros((16,), _i32)
                for w2 in range(_NW):
                    hv = htbl[pl.ds(w2 * bins + jb * 16, 16)]
                    tot = tot + hv
                    mine = mine + jnp.where(w2 < w, hv, 0)
                csum = plsc.cumsum(tot)
                offtbl[pl.ds(jb * 16, 16)] = csum - tot + carry + mine
                return carry + jnp.max(csum)

            _fori(bins // 16, ob, _i32(0))

            def bt(t, _):
                b = base + t * _BK
                pltpu.sync_copy(key_h.at[pl.ds(_al(b), _BK)], kbuf)
                if not first:
                    pltpu.sync_copy(id_h.at[pl.ds(_al(b), _BK)], ibuf)

                def bi(i, _2):
                    kk = kbuf[pl.ds(i * 16, 16)]
                    d = _srl(kk, shift) & _i32(bins - 1)
                    rank, lastm = plsc.scan_count(d)
                    off = plsc.load_gather(offtbl, [d])
                    pbuf[pl.ds(i * 16, 16)] = off + rank - 1
                    plsc.addupdate_scatter(offtbl, [d], rank, mask=lastm)
                    if first:
                        ibuf[pl.ds(i * 16, 16)] = _lane() + (b + i * 16)
                    return 0

                _fori(NV, bi, 0)
                c1 = pltpu.async_copy(kbuf, keyo_h.at[pbuf], sem)
                c2 = pltpu.async_copy(ibuf, ido_h.at[pbuf], sem)
                c1.wait()
                c2.wait()
                return 0

            _fori(NB, bt, 0)

        return pl.kernel(
            body,
            out_type=(jax.ShapeDtypeStruct((M,), _i32),
                      jax.ShapeDtypeStruct((M,), _i32)),
            mesh=_MESH, compiler_params=_CP,
            scratch_types=[pltpu.VMEM((_NW * bins,), _i32),
                           pltpu.VMEM((bins,), _i32),
                           pltpu.VMEM((_BK,), _i32),
                           pltpu.VMEM((_BK,), _i32),
                           pltpu.VMEM((_BK,), _i32),
                           pltpu.SemaphoreType.DMA])

    @functools.partial(
        pl.kernel,
        out_type=jax.ShapeDtypeStruct((_NW * 16,), _i32),
        mesh=_MESH, compiler_params=_CP,
        scratch_types=[pltpu.VMEM((_BK + 16,), _i32),
                       pltpu.VMEM((16,), _i32)])
    def k_count(key_h, nseg_h, kext, sbuf):
        w = _wid()
        base = w * CH
        lane = _lane()

        def bt(t, cnt):
            b = base + t * _BK
            pb = jnp.maximum(b - 16, 0)
            pltpu.sync_copy(key_h.at[pl.ds(_al(pb), 16)], kext.at[pl.ds(0, 16)])
            pltpu.sync_copy(key_h.at[pl.ds(_al(b), _BK)], kext.at[pl.ds(16, _BK)])

            def bi(i, c2):
                x = kext[pl.ds(16 + i * 16, 16)]
                xp = kext[pl.ds(15 + i * 16, 16)]
                fl = (x != xp).astype(_i32)
                g0 = jnp.logical_and(b + i * 16 == 0, lane == 0)
                fl = jnp.where(g0, 1, fl)
                return c2 + jnp.sum(fl, dtype=_i32)

            return _fori(NV, bi, cnt)

        cnt = _fori(NB, bt, _i32(0))
        sbuf[...] = _splat(cnt)
        pltpu.sync_copy(sbuf, nseg_h.at[pl.ds(_al(w * 16), 16)])

    @functools.partial(
        pl.kernel,
        out_type=(jax.ShapeDtypeStruct((M + 8, 16), _f32),
                  jax.ShapeDtypeStruct((M + 8,), _i32),
                  jax.ShapeDtypeStruct((M + 8,), _i32),
                  jax.ShapeDtypeStruct((_NW, 16), _f32),
                  jax.ShapeDtypeStruct((_NW * 16,), _i32),
                  jax.ShapeDtypeStruct((_NW * 16,), _i32)),
        mesh=_MESH, compiler_params=_CP,
        scratch_types=[pltpu.VMEM((_BK + 32,), _i32),
                       pltpu.VMEM((_BK,), _i32),
                       pltpu.VMEM((_BK,), _i32),
                       pltpu.VMEM((_BK, 16), _f32),
                       pltpu.VMEM((_BK,), _i32),
                       pltpu.VMEM((_BK,), _i32),
                       pltpu.VMEM((_BK,), _i32),
                       pltpu.VMEM((_BK,), _i32),
                       pltpu.VMEM((_BK,), _i32),
                       pltpu.VMEM((_BK,), _i32),
                       pltpu.VMEM((_NW * 16,), _i32),
                       pltpu.VMEM((16,), _f32),
                       pltpu.VMEM((16,), _i32),
                       pltpu.SemaphoreType.DMA])
    def k_seg(key_h, id_h, attr_h, nseg_h,
              outa_h, us_h, ud_h, tmax_h, tinv_h, cont_h,
              kext, idb, imb, rows, stf, enf, pose, posu, sbuf, dbuf,
              nbuf, v16f, v16i, sem):
        w = _wid()
        base = w * CH
        lane = _lane()
        pltpu.sync_copy(nseg_h, nbuf)
        s0v = jnp.zeros((16,), _i32)
        totv = jnp.zeros((16,), _i32)
        for w2 in range(_NW):
            v = nbuf[pl.ds(w2 * 16, 16)]
            s0v = s0v + jnp.where(w2 < w, v, 0)
            totv = totv + v
        s0 = jnp.max(s0v)
        U = jnp.max(totv)
        neg = jnp.full((16,), -jnp.inf, _f32)

        def bt(t, carry):
            acc, seg = carry
            b = base + t * _BK
            pb = jnp.maximum(b - 16, 0)
            nb2 = jnp.minimum(b + _BK, M - 16)
            pltpu.sync_copy(key_h.at[pl.ds(_al(pb), 16)], kext.at[pl.ds(0, 16)])
            pltpu.sync_copy(key_h.at[pl.ds(_al(b), _BK)], kext.at[pl.ds(16, _BK)])
            pltpu.sync_copy(key_h.at[pl.ds(_al(nb2), 16)],
                            kext.at[pl.ds(16 + _BK, 16)])
            pltpu.sync_copy(id_h.at[pl.ds(_al(b), _BK)], idb)

            def bi(i, segc):
                kk = kext[pl.ds(16 + i * 16, 16)]
                xp = kext[pl.ds(15 + i * 16, 16)]
                xn = kext[pl.ds(17 + i * 16, 16)]
                st = (kk != xp).astype(_i32)
                g0 = jnp.logical_and(b + i * 16 == 0, lane == 0)
                st = jnp.where(g0, 1, st)
                en = (kk != xn).astype(_i32)
                gl = jnp.logical_and(b + i * 16 + 16 == M, lane == 15)
                en = jnp.where(gl, 1, en)
                stf[pl.ds(i * 16, 16)] = st
                enf[pl.ds(i * 16, 16)] = en
                c = plsc.cumsum(st)
                loc = segc + c - 1
                trash = _i32(M) + (lane & 7)
                posu[pl.ds(i * 16, 16)] = jnp.where(st > 0, s0 + loc, trash)
                pose[pl.ds(i * 16, 16)] = jnp.where(en > 0, s0 + loc, trash)
                sbuf[pl.ds(i * 16, 16)] = _srl(kk, 16)
                dbuf[pl.ds(i * 16, 16)] = kk & _i32(0xFFFF)
                ii = idb[pl.ds(i * 16, 16)]
                imb[pl.ds(i * 16, 16)] = ii - jnp.where(ii >= _i32(E), _i32(E), _i32(0))
                return segc + jnp.max(c)

            seg = _fori(NV, bi, seg)
            pltpu.async_copy(attr_h.at[imb], rows, sem).wait()

            def pe(j, a):
                r = rows[j, :]
                ss = plsc.load_gather(stf, [_splat(j)])
                se = plsc.load_gather(enf, [_splat(j)])
                a2 = jnp.maximum(r, jnp.where(ss > 0, neg, a))
                rows[j, :] = jnp.where(se > 0, a2, r)
                return a2

            acc = _fori(_BK, pe, acc)
            c1 = pltpu.async_copy(rows, outa_h.at[pose], sem)
            c2 = pltpu.async_copy(sbuf, us_h.at[posu], sem)
            c3 = pltpu.async_copy(dbuf, ud_h.at[posu], sem)
            c1.wait()
            c2.wait()
            c3.wait()
            return (acc, seg)

        acc, seg = _fori(NB, bt, (jnp.full((16,), -jnp.inf, _f32), _i32(0)))

        v16f[...] = acc
        pltpu.sync_copy(v16f, tmax_h.at[w])
        v16i[...] = _splat(s0 + seg - 1)
        pltpu.sync_copy(v16i, tinv_h.at[pl.ds(_al(w * 16), 16)])
        eb = jnp.minimum(base + CH, M - 16)
        pltpu.sync_copy(key_h.at[pl.ds(_al(base + CH - 16), 16)],
                        kext.at[pl.ds(0, 16)])
        pltpu.sync_copy(key_h.at[pl.ds(_al(eb), 16)], kext.at[pl.ds(16, 16)])
        x1 = kext[pl.ds(0, 16)]
        x2 = kext[pl.ds(16, 16)]
        e_last = jnp.sum(jnp.where(lane == 15, x1, 0), dtype=_i32)
        e_next = jnp.sum(jnp.where(lane == 0, x2, 0), dtype=_i32)
        cont = jnp.logical_and(w < _NW - 1, e_last == e_next)
        v16i[...] = _splat(cont.astype(_i32))
        pltpu.sync_copy(v16i, cont_h.at[pl.ds(_al(w * 16), 16)])

        lo = jnp.maximum(U, w * CH)
        hi = (w + 1) * CH

        def zr(j, _):
            rows[j, :] = jnp.zeros((16,), _f32)
            return 0

        _fori(_BK, zr, 0)

        def fv(i, _):
            sbuf[pl.ds(i * 16, 16)] = jnp.full((16,), -1, _i32)
            dbuf[pl.ds(i * 16, 16)] = jnp.full((16,), _N - 1, _i32)
            return 0

        _fori(NV, fv, 0)
        nblk = jnp.maximum(0, (hi - lo + _BK - 1) // _BK)

        def fb(q, _):
            st = lo + q * _BK

            def fi(i, _2):
                v = jnp.minimum(st + i * 16 + lane, hi - 1)
                pose[pl.ds(i * 16, 16)] = v
                return 0

            _fori(NV, fi, 0)
            c1 = pltpu.async_copy(rows, outa_h.at[pose], sem)
            c2 = pltpu.async_copy(sbuf, us_h.at[pose], sem)
            c3 = pltpu.async_copy(dbuf, ud_h.at[pose], sem)
            c1.wait()
            c2.wait()
            c3.wait()
            return 0

        _fori(nblk, fb, 0)

    hists = [make_hist(s, 1 << b) for s, b in zip(_SHIFTS, _BITS)]
    scats = [make_scat(s, 1 << b, i == 0)
             for i, (s, b) in enumerate(zip(_SHIFTS, _BITS))]
    return k_build, hists, scats, k_count, k_seg


def kernel(edge_index, edge_attr):
    E = edge_index.shape[1]
    M = 2 * E
    k_build, hists, scats, k_count, k_seg = _make_kernels(E)

    ei = edge_index.astype(jnp.int32)
    row, col = ei[0], ei[1]
    attr = edge_attr.astype(jnp.float32)

    key = k_build(row, col)
    h = hists[0](key)
    key, ids = scats[0](key, h)
    h = hists[1](key)
    key, ids = scats[1](key, ids, h)
    h = hists[2](key)
    key, ids = scats[2](key, ids, h)
    nseg = k_count(key)
    outa, us, ud, tmax, tinv, cont = k_seg(key, ids, attr, nseg)

    tinv0 = tinv.reshape(_NW, 16)[:, 0]
    cont0 = cont.reshape(_NW, 16)[:, 0]
    prow = jnp.where(cont0 > 0, tinv0, jnp.int32(M))
    outa = outa.at[prow].max(tmax)

    out = outa[:M]
    src = us[:M].astype(jnp.int64)
    dst = ud[:M].astype(jnp.int64)
    new_edge_index = jnp.stack([src, dst])
    return new_edge_index, out


# speedup vs baseline: 26.3739x; 1.0032x over previous
import functools

import jax
import jax.numpy as jnp
from jax import lax
from jax.experimental import pallas as pl
from jax.experimental.pallas import tpu as pltpu
from jax.experimental.pallas import tpu_sc as plsc

_N = 50000
_NC, _NS = 2, 16
_NW = _NC * _NS
_BITS = (11, 11, 10)
_SHIFTS = (0, 11, 22)
_BK = 2000

_MESH = plsc.VectorSubcoreMesh(core_axis_name="c", subcore_axis_name="s")
_CP = pltpu.CompilerParams(
    needs_layout_passes=False, use_tc_tiling_on_sc=False)

_i32 = jnp.int32
_f32 = jnp.float32


def _wid():
    return lax.axis_index("s") * _NC + lax.axis_index("c")


def _lane():
    return lax.iota(_i32, 16)


def _srl(x, s):
    return lax.shift_right_logical(x, jnp.full((16,), s, _i32))


def _sll(x, s):
    return lax.shift_left(x, jnp.full((16,), s, _i32))


def _splat(x):
    return jnp.zeros((16,), _i32) + x


def _fori(n, body, init):
    return lax.fori_loop(_i32(0), _i32(n), body, init)


def _al(x):
    return pl.multiple_of(x, 8)


def _make_kernels(E):
    M = 2 * E
    CH = M // _NW
    assert CH % _BK == 0 and E % CH == 0
    NB = CH // _BK
    NV = _BK // 16

    @functools.partial(
        pl.kernel,
        out_type=jax.ShapeDtypeStruct((M,), _i32),
        mesh=_MESH, compiler_params=_CP,
        scratch_types=[pltpu.VMEM((_BK,), _i32),
                       pltpu.VMEM((_BK,), _i32),
                       pltpu.VMEM((_BK,), _i32)])
    def k_build(row_h, col_h, key_h, rbuf, cbuf, kbuf):
        base = _wid() * CH

        def bt(t, _):
            b = base + t * _BK
            flip = b >= E
            eb = jnp.where(flip, b - E, b)
            pltpu.sync_copy(row_h.at[pl.ds(_al(eb), _BK)], rbuf)
            pltpu.sync_copy(col_h.at[pl.ds(_al(eb), _BK)], cbuf)

            def bi(i, _2):
                r = rbuf[pl.ds(i * 16, 16)]
                c = cbuf[pl.ds(i * 16, 16)]
                fwd = _sll(r, 16) | c
                rev = _sll(c, 16) | r
                kbuf[pl.ds(i * 16, 16)] = jnp.where(flip, rev, fwd)
                return 0

            _fori(NV, bi, 0)
            pltpu.sync_copy(kbuf, key_h.at[pl.ds(_al(b), _BK)])
            return 0

        _fori(NB, bt, 0)

    def make_hist(shift, bins):
        @functools.partial(
            pl.kernel,
            out_type=jax.ShapeDtypeStruct((_NW * bins,), _i32),
            mesh=_MESH, compiler_params=_CP,
            scratch_types=[pltpu.VMEM((_BK,), _i32),
                           pltpu.VMEM((bins,), _i32)])
        def k_hist(key_h, hist_h, kbuf, htbl):
            w = _wid()
            base = w * CH

            def z(j, _):
                htbl[pl.ds(j * 16, 16)] = jnp.zeros((16,), _i32)
                return 0

            _fori(bins // 16, z, 0)

            def bt(t, _):
                pltpu.sync_copy(key_h.at[pl.ds(_al(base + t * _BK), _BK)], kbuf)

                def bi(i, _2):
                    kk = kbuf[pl.ds(i * 16, 16)]
                    d = _srl(kk, shift) & _i32(bins - 1)
                    rank, lastm = plsc.scan_count(d)
                    plsc.addupdate_scatter(htbl, [d], rank, mask=lastm)
                    return 0

                _fori(NV, bi, 0)
                return 0

            _fori(NB, bt, 0)
            pltpu.sync_copy(htbl, hist_h.at[pl.ds(_al(w * bins), bins)])

        return k_hist

    def make_scat(shift, bins, first):
        ins = 2 if first else 3

        def body(*refs):
            if first:
                key_h, hist_h = refs[0], refs[1]
                id_h = None
                keyo_h, ido_h = refs[2], refs[3]
                htbl, offtbl, kbuf, ibuf, pbuf, sem = refs[4:]
            else:
                key_h, id_h, hist_h = refs[0], refs[1], refs[2]
                keyo_h, ido_h = refs[3], refs[4]
                htbl, offtbl, kbuf, ibuf, pbuf, sem = refs[5:]
            w = _wid()
            base = w * CH
            pltpu.sync_copy(hist_h, htbl)

            def ob(jb, carry):
                tot = jnp.zeros((16,), _i32)
                mine = jnp.zeros((16,), _i32)
                for w2 in range(_NW):
                    hv = htbl[pl.ds(w2 * bins + jb * 16, 16)]
                    tot = tot + hv
                    mine = mine + jnp.where(w2 < w, hv, 0)
                csum = plsc.cumsum(tot)
                offtbl[pl.ds(jb * 16, 16)] = csum - tot + carry + mine
                return carry + jnp.max(csum)

            _fori(bins // 16, ob, _i32(0))

            def bt(t, _):
                b = base + t * _BK
                pltpu.sync_copy(key_h.at[pl.ds(_al(b), _BK)], kbuf)
                if not first:
                    pltpu.sync_copy(id_h.at[pl.ds(_al(b), _BK)], ibuf)

                def bi(i, _2):
                    kk = kbuf[pl.ds(i * 16, 16)]
                    d = _srl(kk, shift) & _i32(bins - 1)
                    rank, lastm = plsc.scan_count(d)
                    off = plsc.load_gather(offtbl, [d])
                    pbuf[pl.ds(i * 16, 16)] = off + rank - 1
                    plsc.addupdate_scatter(offtbl, [d], rank, mask=lastm)
                    if first:
                        ibuf[pl.ds(i * 16, 16)] = _lane() + (b + i * 16)
                    return 0

                _fori(NV, bi, 0)
                c1 = pltpu.async_copy(kbuf, keyo_h.at[pbuf], sem)
                c2 = pltpu.async_copy(ibuf, ido_h.at[pbuf], sem)
                c1.wait()
                c2.wait()
                return 0

            _fori(NB, bt, 0)

        return pl.kernel(
            body,
            out_type=(jax.ShapeDtypeStruct((M,), _i32),
                      jax.ShapeDtypeStruct((M,), _i32)),
            mesh=_MESH, compiler_params=_CP,
            scratch_types=[pltpu.VMEM((_NW * bins,), _i32),
                           pltpu.VMEM((bins,), _i32),
                           pltpu.VMEM((_BK,), _i32),
                           pltpu.VMEM((_BK,), _i32),
                           pltpu.VMEM((_BK,), _i32),
                           pltpu.SemaphoreType.DMA])

    @functools.partial(
        pl.kernel,
        out_type=jax.ShapeDtypeStruct((_NW * 16,), _i32),
        mesh=_MESH, compiler_params=_CP,
        scratch_types=[pltpu.VMEM((_BK + 16,), _i32),
                       pltpu.VMEM((16,), _i32)])
    def k_count(key_h, nseg_h, kext, sbuf):
        w = _wid()
        base = w * CH
        lane = _lane()

        def bt(t, cnt):
            b = base + t * _BK
            pb = jnp.maximum(b - 16, 0)
            pltpu.sync_copy(key_h.at[pl.ds(_al(pb), 16)], kext.at[pl.ds(0, 16)])
            pltpu.sync_copy(key_h.at[pl.ds(_al(b), _BK)], kext.at[pl.ds(16, _BK)])

            def bi(i, c2):
                x = kext[pl.ds(16 + i * 16, 16)]
                xp = kext[pl.ds(15 + i * 16, 16)]
                fl = (x != xp).astype(_i32)
                g0 = jnp.logical_and(b + i * 16 == 0, lane == 0)
                fl = jnp.where(g0, 1, fl)
                return c2 + jnp.sum(fl, dtype=_i32)

            return _fori(NV, bi, cnt)

        cnt = _fori(NB, bt, _i32(0))
        sbuf[...] = _splat(cnt)
        pltpu.sync_copy(sbuf, nseg_h.at[pl.ds(_al(w * 16), 16)])

    @functools.partial(
        pl.kernel,
        out_type=(jax.ShapeDtypeStruct((M + 8, 16), _f32),
                  jax.ShapeDtypeStruct((M + 8,), _i32),
                  jax.ShapeDtypeStruct((M + 8,), _i32),
                  jax.ShapeDtypeStruct((_NW, 16), _f32),
                  jax.ShapeDtypeStruct((_NW * 16,), _i32),
                  jax.ShapeDtypeStruct((_NW * 16,), _i32)),
        mesh=_MESH, compiler_params=_CP,
        scratch_types=[pltpu.VMEM((_BK + 32,), _i32),
                       pltpu.VMEM((_BK,), _i32),
                       pltpu.VMEM((_BK,), _i32),
                       pltpu.VMEM((_BK, 16), _f32),
                       pltpu.VMEM((_BK,), _i32),
                       pltpu.VMEM((_BK,), _i32),
                       pltpu.VMEM((_BK,), _i32),
                       pltpu.VMEM((_BK,), _i32),
                       pltpu.VMEM((_BK,), _i32),
                       pltpu.VMEM((_BK,), _i32),
                       pltpu.VMEM((_NW * 16,), _i32),
                       pltpu.VMEM((16,), _f32),
                       pltpu.VMEM((16,), _i32),
                       pltpu.SemaphoreType.DMA])
    def k_seg(key_h, id_h, attr_h, nseg_h,
              outa_h, us_h, ud_h, tmax_h, tinv_h, cont_h,
              kext, idb, imb, rows, stf, enf, pose, posu, sbuf, dbuf,
              nbuf, v16f, v16i, sem):
        w = _wid()
        base = w * CH
        lane = _lane()
        pltpu.sync_copy(nseg_h, nbuf)
        s0v = jnp.zeros((16,), _i32)
        totv = jnp.zeros((16,), _i32)
        for w2 in range(_NW):
            v = nbuf[pl.ds(w2 * 16, 16)]
            s0v = s0v + jnp.where(w2 < w, v, 0)
            totv = totv + v
        s0 = jnp.max(s0v)
        U = jnp.max(totv)
        neg = jnp.full((16,), -jnp.inf, _f32)

        def bt(t, carry):
            acc, seg = carry
            b = base + t * _BK
            pb = jnp.maximum(b - 16, 0)
            nb2 = jnp.minimum(b + _BK, M - 16)
            pltpu.sync_copy(key_h.at[pl.ds(_al(pb), 16)], kext.at[pl.ds(0, 16)])
            pltpu.sync_copy(key_h.at[pl.ds(_al(b), _BK)], kext.at[pl.ds(16, _BK)])
            pltpu.sync_copy(key_h.at[pl.ds(_al(nb2), 16)],
                            kext.at[pl.ds(16 + _BK, 16)])
            pltpu.sync_copy(id_h.at[pl.ds(_al(b), _BK)], idb)

            def bi(i, segc):
                kk = kext[pl.ds(16 + i * 16, 16)]
                xp = kext[pl.ds(15 + i * 16, 16)]
                xn = kext[pl.ds(17 + i * 16, 16)]
                st = (kk != xp).astype(_i32)
                g0 = jnp.logical_and(b + i * 16 == 0, lane == 0)
                st = jnp.where(g0, 1, st)
                en = (kk != xn).astype(_i32)
                gl = jnp.logical_and(b + i * 16 + 16 == M, lane == 15)
                en = jnp.where(gl, 1, en)
                stf[pl.ds(i * 16, 16)] = st
                enf[pl.ds(i * 16, 16)] = en
                c = plsc.cumsum(st)
                loc = segc + c - 1
                trash = _i32(M) + (lane & 7)
                posu[pl.ds(i * 16, 16)] = jnp.where(st > 0, s0 + loc, trash)
                pose[pl.ds(i * 16, 16)] = jnp.where(en > 0, s0 + loc, trash)
                sbuf[pl.ds(i * 16, 16)] = _srl(kk, 16)
                dbuf[pl.ds(i * 16, 16)] = kk & _i32(0xFFFF)
                ii = idb[pl.ds(i * 16, 16)]
                imb[pl.ds(i * 16, 16)] = ii - jnp.where(ii >= _i32(E), _i32(E), _i32(0))
                return segc + jnp.max(c)

            seg = _fori(NV, bi, seg)
            pltpu.async_copy(attr_h.at[imb], rows, sem).wait()

            def pe(j, a):
                r = rows[j, :]
                ss = plsc.load_gather(stf, [_splat(j)])
                se = plsc.load_gather(enf, [_splat(j)])
                a2 = jnp.maximum(r, jnp.where(ss > 0, neg, a))
                rows[j, :] = jnp.where(se > 0, a2, r)
                return a2

            def pv(i, a):
                stv = stf[pl.ds(i * 16, 16)]
                env = enf[pl.ds(i * 16, 16)]
                allf = jnp.sum(stv + env, dtype=_i32) == 32

                def slow(a0):
                    return lax.fori_loop(i * 16, i * 16 + 16, pe, a0)

                return lax.cond(allf, lambda a0: a0, slow, a)

            acc = _fori(NV, pv, acc)
            c1 = pltpu.async_copy(rows, outa_h.at[pose], sem)
            c2 = pltpu.async_copy(sbuf, us_h.at[posu], sem)
            c3 = pltpu.async_copy(dbuf, ud_h.at[posu], sem)
            c1.wait()
            c2.wait()
            c3.wait()
            return (acc, seg)

        acc, seg = _fori(NB, bt, (jnp.full((16,), -jnp.inf, _f32), _i32(0)))

        v16f[...] = acc
        pltpu.sync_copy(v16f, tmax_h.at[w])
        v16i[...] = _splat(s0 + seg - 1)
        pltpu.sync_copy(v16i, tinv_h.at[pl.ds(_al(w * 16), 16)])
        eb = jnp.minimum(base + CH, M - 16)
        pltpu.sync_copy(key_h.at[pl.ds(_al(base + CH - 16), 16)],
                        kext.at[pl.ds(0, 16)])
        pltpu.sync_copy(key_h.at[pl.ds(_al(eb), 16)], kext.at[pl.ds(16, 16)])
        x1 = kext[pl.ds(0, 16)]
        x2 = kext[pl.ds(16, 16)]
        e_last = jnp.sum(jnp.where(lane == 15, x1, 0), dtype=_i32)
        e_next = jnp.sum(jnp.where(lane == 0, x2, 0), dtype=_i32)
        cont = jnp.logical_and(w < _NW - 1, e_last == e_next)
        v16i[...] = _splat(cont.astype(_i32))
        pltpu.sync_copy(v16i, cont_h.at[pl.ds(_al(w * 16), 16)])

        lo = jnp.maximum(U, w * CH)
        hi = (w + 1) * CH

        def zr(j, _):
            rows[j, :] = jnp.zeros((16,), _f32)
            return 0

        _fori(_BK, zr, 0)

        def fv(i, _):
            sbuf[pl.ds(i * 16, 16)] = jnp.full((16,), -1, _i32)
            dbuf[pl.ds(i * 16, 16)] = jnp.full((16,), _N - 1, _i32)
            return 0

        _fori(NV, fv, 0)
        nblk = jnp.maximum(0, (hi - lo + _BK - 1) // _BK)

        def fb(q, _):
            st = lo + q * _BK

            def fi(i, _2):
                v = jnp.minimum(st + i * 16 + lane, hi - 1)
                pose[pl.ds(i * 16, 16)] = v
                return 0

            _fori(NV, fi, 0)
            c1 = pltpu.async_copy(rows, outa_h.at[pose], sem)
            c2 = pltpu.async_copy(sbuf, us_h.at[pose], sem)
            c3 = pltpu.async_copy(dbuf, ud_h.at[pose], sem)
            c1.wait()
            c2.wait()
            c3.wait()
            return 0

        _fori(nblk, fb, 0)

    hists = [make_hist(s, 1 << b) for s, b in zip(_SHIFTS, _BITS)]
    scats = [make_scat(s, 1 << b, i == 0)
             for i, (s, b) in enumerate(zip(_SHIFTS, _BITS))]
    return k_build, hists, scats, k_count, k_seg


def kernel(edge_index, edge_attr):
    E = edge_index.shape[1]
    M = 2 * E
    k_build, hists, scats, k_count, k_seg = _make_kernels(E)

    ei = edge_index.astype(jnp.int32)
    row, col = ei[0], ei[1]
    attr = edge_attr.astype(jnp.float32)

    key = k_build(row, col)
    h = hists[0](key)
    key, ids = scats[0](key, h)
    h = hists[1](key)
    key, ids = scats[1](key, ids, h)
    h = hists[2](key)
    key, ids = scats[2](key, ids, h)
    nseg = k_count(key)
    outa, us, ud, tmax, tinv, cont = k_seg(key, ids, attr, nseg)

    tinv0 = tinv.reshape(_NW, 16)[:, 0]
    cont0 = cont.reshape(_NW, 16)[:, 0]
    prow = jnp.where(cont0 > 0, tinv0, jnp.int32(M))
    outa = outa.at[prow].max(tmax)

    out = outa[:M]
    src = us[:M].astype(jnp.int64)
    dst = ud[:M].astype(jnp.int64)
    new_edge_index = jnp.stack([src, dst])
    return new_edge_index, out


# speedup vs baseline: 26.4468x; 1.0028x over previous
import functools

import jax
import jax.numpy as jnp
from jax import lax
from jax.experimental import pallas as pl
from jax.experimental.pallas import tpu as pltpu
from jax.experimental.pallas import tpu_sc as plsc

_N = 50000
_NC, _NS = 2, 16
_NW = _NC * _NS
_BITS = (11, 11, 10)
_SHIFTS = (0, 11, 22)
_BK = 2000

_MESH = plsc.VectorSubcoreMesh(core_axis_name="c", subcore_axis_name="s")
_CP = pltpu.CompilerParams(
    needs_layout_passes=False, use_tc_tiling_on_sc=False)

_i32 = jnp.int32
_f32 = jnp.float32


def _wid():
    return lax.axis_index("s") * _NC + lax.axis_index("c")


def _lane():
    return lax.iota(_i32, 16)


def _srl(x, s):
    return lax.shift_right_logical(x, jnp.full((16,), s, _i32))


def _sll(x, s):
    return lax.shift_left(x, jnp.full((16,), s, _i32))


def _splat(x):
    return jnp.zeros((16,), _i32) + x


def _fori(n, body, init):
    return lax.fori_loop(_i32(0), _i32(n), body, init)


def _al(x):
    return pl.multiple_of(x, 8)


def _make_kernels(E):
    M = 2 * E
    CH = M // _NW
    assert CH % _BK == 0 and E % CH == 0
    NB = CH // _BK
    NV = _BK // 16

    @functools.partial(
        pl.kernel,
        out_type=jax.ShapeDtypeStruct((M,), _i32),
        mesh=_MESH, compiler_params=_CP,
        scratch_types=[pltpu.VMEM((_BK,), _i32),
                       pltpu.VMEM((_BK,), _i32),
                       pltpu.VMEM((_BK,), _i32)])
    def k_build(row_h, col_h, key_h, rbuf, cbuf, kbuf):
        base = _wid() * CH

        def bt(t, _):
            b = base + t * _BK
            flip = b >= E
            eb = jnp.where(flip, b - E, b)
            pltpu.sync_copy(row_h.at[pl.ds(_al(eb), _BK)], rbuf)
            pltpu.sync_copy(col_h.at[pl.ds(_al(eb), _BK)], cbuf)

            def bi(i, _2):
                r = rbuf[pl.ds(i * 16, 16)]
                c = cbuf[pl.ds(i * 16, 16)]
                fwd = _sll(r, 16) | c
                rev = _sll(c, 16) | r
                kbuf[pl.ds(i * 16, 16)] = jnp.where(flip, rev, fwd)
                return 0

            _fori(NV, bi, 0)
            pltpu.sync_copy(kbuf, key_h.at[pl.ds(_al(b), _BK)])
            return 0

        _fori(NB, bt, 0)

    def make_hist(shift, bins):
        @functools.partial(
            pl.kernel,
            out_type=jax.ShapeDtypeStruct((_NW * bins,), _i32),
            mesh=_MESH, compiler_params=_CP,
            scratch_types=[pltpu.VMEM((_BK,), _i32),
                           pltpu.VMEM((bins,), _i32)])
        def k_hist(key_h, hist_h, kbuf, htbl):
            w = _wid()
            base = w * CH

            def z(j, _):
                htbl[pl.ds(j * 16, 16)] = jnp.zeros((16,), _i32)
                return 0

            _fori(bins // 16, z, 0)

            def bt(t, _):
                pltpu.sync_copy(key_h.at[pl.ds(_al(base + t * _BK), _BK)], kbuf)

                def bi(i, _2):
                    kk = kbuf[pl.ds(i * 16, 16)]
                    d = _srl(kk, shift) & _i32(bins - 1)
                    rank, lastm = plsc.scan_count(d)
                    plsc.addupdate_scatter(htbl, [d], rank, mask=lastm)
                    return 0

                _fori(NV, bi, 0)
                return 0

            _fori(NB, bt, 0)
            pltpu.sync_copy(htbl, hist_h.at[pl.ds(_al(w * bins), bins)])

        return k_hist

    def make_scat(shift, bins, first):
        ins = 2 if first else 3

        def body(*refs):
            if first:
                key_h, hist_h = refs[0], refs[1]
                id_h = None
                keyo_h, ido_h = refs[2], refs[3]
                htbl, offtbl, kbuf, ibuf, pbuf, sem = refs[4:]
            else:
                key_h, id_h, hist_h = refs[0], refs[1], refs[2]
                keyo_h, ido_h = refs[3], refs[4]
                htbl, offtbl, kbuf, ibuf, pbuf, sem = refs[5:]
            w = _wid()
            base = w * CH
            pltpu.sync_copy(hist_h, htbl)

            def ob(jb, carry):
                tot = jnp.zeros((16,), _i32)
                mine = jnp.zeros((16,), _i32)
                for w2 in range(_NW):
                    hv = htbl[pl.ds(w2 * bins + jb * 16, 16)]
                    tot = tot + hv
                    mine = mine + jnp.where(w2 < w, hv, 0)
                csum = plsc.cumsum(tot)
                offtbl[pl.ds(jb * 16, 16)] = csum - tot + carry + mine
                return carry + jnp.max(csum)

            _fori(bins // 16, ob, _i32(0))

            def bt(t, _):
                b = base + t * _BK
                pltpu.sync_copy(key_h.at[pl.ds(_al(b), _BK)], kbuf)
                if not first:
                    pltpu.sync_copy(id_h.at[pl.ds(_al(b), _BK)], ibuf)

                def bi(i, _2):
                    kk = kbuf[pl.ds(i * 16, 16)]
                    d = _srl(kk, shift) & _i32(bins - 1)
                    rank, lastm = plsc.scan_count(d)
                    off = plsc.load_gather(offtbl, [d])
                    pbuf[pl.ds(i * 16, 16)] = off + rank - 1
                    plsc.addupdate_scatter(offtbl, [d], rank, mask=lastm)
                    if first:
                        ibuf[pl.ds(i * 16, 16)] = _lane() + (b + i * 16)
                    return 0

                _fori(NV, bi, 0)
                c1 = pltpu.async_copy(kbuf, keyo_h.at[pbuf], sem)
                c2 = pltpu.async_copy(ibuf, ido_h.at[pbuf], sem)
                c1.wait()
                c2.wait()
                return 0

            _fori(NB, bt, 0)

        return pl.kernel(
            body,
            out_type=(jax.ShapeDtypeStruct((M,), _i32),
                      jax.ShapeDtypeStruct((M,), _i32)),
            mesh=_MESH, compiler_params=_CP,
            scratch_types=[pltpu.VMEM((_NW * bins,), _i32),
                           pltpu.VMEM((bins,), _i32),
                           pltpu.VMEM((_BK,), _i32),
                           pltpu.VMEM((_BK,), _i32),
                           pltpu.VMEM((_BK,), _i32),
                           pltpu.SemaphoreType.DMA])

    @functools.partial(
        pl.kernel,
        out_type=jax.ShapeDtypeStruct((_NW * 16,), _i32),
        mesh=_MESH, compiler_params=_CP,
        scratch_types=[pltpu.VMEM((_BK + 16,), _i32),
                       pltpu.VMEM((16,), _i32)])
    def k_count(key_h, nseg_h, kext, sbuf):
        w = _wid()
        base = w * CH
        lane = _lane()

        def bt(t, cnt):
            b = base + t * _BK
            pb = jnp.maximum(b - 16, 0)
            pltpu.sync_copy(key_h.at[pl.ds(_al(pb), 16)], kext.at[pl.ds(0, 16)])
            pltpu.sync_copy(key_h.at[pl.ds(_al(b), _BK)], kext.at[pl.ds(16, _BK)])

            def bi(i, c2):
                x = kext[pl.ds(16 + i * 16, 16)]
                xp = kext[pl.ds(15 + i * 16, 16)]
                fl = (x != xp).astype(_i32)
                g0 = jnp.logical_and(b + i * 16 == 0, lane == 0)
                fl = jnp.where(g0, 1, fl)
                return c2 + jnp.sum(fl, dtype=_i32)

            return _fori(NV, bi, cnt)

        cnt = _fori(NB, bt, _i32(0))
        sbuf[...] = _splat(cnt)
        pltpu.sync_copy(sbuf, nseg_h.at[pl.ds(_al(w * 16), 16)])

    @functools.partial(
        pl.kernel,
        out_type=(jax.ShapeDtypeStruct((M + 8, 16), _f32),
                  jax.ShapeDtypeStruct((M + 8,), _i32),
                  jax.ShapeDtypeStruct((M + 8,), _i32),
                  jax.ShapeDtypeStruct((_NW, 16), _f32),
                  jax.ShapeDtypeStruct((_NW * 16,), _i32),
                  jax.ShapeDtypeStruct((_NW * 16,), _i32)),
        mesh=_MESH, compiler_params=_CP,
        scratch_types=[pltpu.VMEM((_BK + 32,), _i32),
                       pltpu.VMEM((_BK,), _i32),
                       pltpu.VMEM((_BK,), _i32),
                       pltpu.VMEM((_BK, 16), _f32),
                       pltpu.VMEM((_BK,), _i32),
                       pltpu.VMEM((_BK,), _i32),
                       pltpu.VMEM((_BK,), _i32),
                       pltpu.VMEM((_BK,), _i32),
                       pltpu.VMEM((_BK,), _i32),
                       pltpu.VMEM((_BK,), _i32),
                       pltpu.VMEM((_NW * 16,), _i32),
                       pltpu.VMEM((16,), _f32),
                       pltpu.VMEM((16,), _i32),
                       pltpu.SemaphoreType.DMA])
    def k_seg(key_h, id_h, attr_h, nseg_h,
              outa_h, us_h, ud_h, tmax_h, tinv_h, cont_h,
              kext, idb, imb, rows, stf, enf, pose, posu, sbuf, dbuf,
              nbuf, v16f, v16i, sem):
        w = _wid()
        base = w * CH
        lane = _lane()
        pltpu.sync_copy(nseg_h, nbuf)
        s0v = jnp.zeros((16,), _i32)
        totv = jnp.zeros((16,), _i32)
        for w2 in range(_NW):
            v = nbuf[pl.ds(w2 * 16, 16)]
            s0v = s0v + jnp.where(w2 < w, v, 0)
            totv = totv + v
        s0 = jnp.max(s0v)
        U = jnp.max(totv)
        neg = jnp.full((16,), -jnp.inf, _f32)

        def bt(t, carry):
            acc, seg = carry
            b = base + t * _BK
            pb = jnp.maximum(b - 16, 0)
            nb2 = jnp.minimum(b + _BK, M - 16)
            pltpu.sync_copy(key_h.at[pl.ds(_al(pb), 16)], kext.at[pl.ds(0, 16)])
            pltpu.sync_copy(key_h.at[pl.ds(_al(b), _BK)], kext.at[pl.ds(16, _BK)])
            pltpu.sync_copy(key_h.at[pl.ds(_al(nb2), 16)],
                            kext.at[pl.ds(16 + _BK, 16)])
            pltpu.sync_copy(id_h.at[pl.ds(_al(b), _BK)], idb)

            def bi(i, cc):
                segc, fsum = cc
                kk = kext[pl.ds(16 + i * 16, 16)]
                xp = kext[pl.ds(15 + i * 16, 16)]
                xn = kext[pl.ds(17 + i * 16, 16)]
                st = (kk != xp).astype(_i32)
                g0 = jnp.logical_and(b + i * 16 == 0, lane == 0)
                st = jnp.where(g0, 1, st)
                en = (kk != xn).astype(_i32)
                gl = jnp.logical_and(b + i * 16 + 16 == M, lane == 15)
                en = jnp.where(gl, 1, en)
                stf[pl.ds(i * 16, 16)] = st
                enf[pl.ds(i * 16, 16)] = en
                c = plsc.cumsum(st)
                loc = segc + c - 1
                trash = _i32(M) + (lane & 7)
                posu[pl.ds(i * 16, 16)] = jnp.where(st > 0, s0 + loc, trash)
                pose[pl.ds(i * 16, 16)] = jnp.where(en > 0, s0 + loc, trash)
                sbuf[pl.ds(i * 16, 16)] = _srl(kk, 16)
                dbuf[pl.ds(i * 16, 16)] = kk & _i32(0xFFFF)
                ii = idb[pl.ds(i * 16, 16)]
                imb[pl.ds(i * 16, 16)] = ii - jnp.where(ii >= _i32(E), _i32(E), _i32(0))
                return (segc + jnp.max(c),
                        fsum + jnp.sum(st + en, dtype=_i32))

            segbase = seg
            seg, fsum = _fori(NV, bi, (seg, _i32(0)))
            pltpu.async_copy(attr_h.at[imb], rows, sem).wait()

            def pe(j, a):
                r = rows[j, :]
                ss = plsc.load_gather(stf, [_splat(j)])
                se = plsc.load_gather(enf, [_splat(j)])
                a2 = jnp.maximum(r, jnp.where(ss > 0, neg, a))
                rows[j, :] = jnp.where(se > 0, a2, r)
                return a2

            c2 = pltpu.async_copy(sbuf, us_h.at[posu], sem)
            c3 = pltpu.async_copy(dbuf, ud_h.at[posu], sem)

            def fastb(a):
                pltpu.sync_copy(rows, outa_h.at[pl.ds(s0 + segbase, _BK)])
                return a

            def slowb(a):
                def pv(i, a0):
                    stv = stf[pl.ds(i * 16, 16)]
                    env = enf[pl.ds(i * 16, 16)]
                    allf = jnp.sum(stv + env, dtype=_i32) == 32

                    def slow(a1):
                        return lax.fori_loop(i * 16, i * 16 + 16, pe, a1)

                    return lax.cond(allf, lambda a1: a1, slow, a0)

                a = _fori(NV, pv, a)
                pltpu.async_copy(rows, outa_h.at[pose], sem).wait()
                return a

            acc = lax.cond(fsum == 2 * _BK, fastb, slowb, acc)
            c2.wait()
            c3.wait()
            return (acc, seg)

        acc, seg = _fori(NB, bt, (jnp.full((16,), -jnp.inf, _f32), _i32(0)))

        v16f[...] = acc
        pltpu.sync_copy(v16f, tmax_h.at[w])
        v16i[...] = _splat(s0 + seg - 1)
        pltpu.sync_copy(v16i, tinv_h.at[pl.ds(_al(w * 16), 16)])
        eb = jnp.minimum(base + CH, M - 16)
        pltpu.sync_copy(key_h.at[pl.ds(_al(base + CH - 16), 16)],
                        kext.at[pl.ds(0, 16)])
        pltpu.sync_copy(key_h.at[pl.ds(_al(eb), 16)], kext.at[pl.ds(16, 16)])
        x1 = kext[pl.ds(0, 16)]
        x2 = kext[pl.ds(16, 16)]
        e_last = jnp.sum(jnp.where(lane == 15, x1, 0), dtype=_i32)
        e_next = jnp.sum(jnp.where(lane == 0, x2, 0), dtype=_i32)
        cont = jnp.logical_and(w < _NW - 1, e_last == e_next)
        v16i[...] = _splat(cont.astype(_i32))
        pltpu.sync_copy(v16i, cont_h.at[pl.ds(_al(w * 16), 16)])

        lo = jnp.maximum(U, w * CH)
        hi = (w + 1) * CH

        def zr(j, _):
            rows[j, :] = jnp.zeros((16,), _f32)
            return 0

        _fori(_BK, zr, 0)

        def fv(i, _):
            sbuf[pl.ds(i * 16, 16)] = jnp.full((16,), -1, _i32)
            dbuf[pl.ds(i * 16, 16)] = jnp.full((16,), _N - 1, _i32)
            return 0

        _fori(NV, fv, 0)
        nblk = jnp.maximum(0, (hi - lo + _BK - 1) // _BK)

        def fb(q, _):
            st = lo + q * _BK

            def fi(i, _2):
                v = jnp.minimum(st + i * 16 + lane, hi - 1)
                pose[pl.ds(i * 16, 16)] = v
                return 0

            _fori(NV, fi, 0)
            c1 = pltpu.async_copy(rows, outa_h.at[pose], sem)
            c2 = pltpu.async_copy(sbuf, us_h.at[pose], sem)
            c3 = pltpu.async_copy(dbuf, ud_h.at[pose], sem)
            c1.wait()
            c2.wait()
            c3.wait()
            return 0

        _fori(nblk, fb, 0)

    hists = [make_hist(s, 1 << b) for s, b in zip(_SHIFTS, _BITS)]
    scats = [make_scat(s, 1 << b, i == 0)
             for i, (s, b) in enumerate(zip(_SHIFTS, _BITS))]
    return k_build, hists, scats, k_count, k_seg


def kernel(edge_index, edge_attr):
    E = edge_index.shape[1]
    M = 2 * E
    k_build, hists, scats, k_count, k_seg = _make_kernels(E)

    ei = edge_index.astype(jnp.int32)
    row, col = ei[0], ei[1]
    attr = edge_attr.astype(jnp.float32)

    key = k_build(row, col)
    h = hists[0](key)
    key, ids = scats[0](key, h)
    h = hists[1](key)
    key, ids = scats[1](key, ids, h)
    h = hists[2](key)
    key, ids = scats[2](key, ids, h)
    nseg = k_count(key)
    outa, us, ud, tmax, tinv, cont = k_seg(key, ids, attr, nseg)

    tinv0 = tinv.reshape(_NW, 16)[:, 0]
    cont0 = cont.reshape(_NW, 16)[:, 0]
    prow = jnp.where(cont0 > 0, tinv0, jnp.int32(M))
    outa = outa.at[prow].max(tmax)

    out = outa[:M]
    src = us[:M].astype(jnp.int64)
    dst = ud[:M].astype(jnp.int64)
    new_edge_index = jnp.stack([src, dst])
    return new_edge_index, out


# speedup vs baseline: 26.5087x; 1.0023x over previous
import functools

import jax
import jax.numpy as jnp
from jax import lax
from jax.experimental import pallas as pl
from jax.experimental.pallas import tpu as pltpu
from jax.experimental.pallas import tpu_sc as plsc

_N = 50000
_NC, _NS = 2, 16
_NW = _NC * _NS
_BITS = (11, 11, 10)
_SHIFTS = (0, 11, 22)
_BK = 2000
_SS = 5
_SB = _BK // _SS
_SV = _SB // 16

_MESH = plsc.VectorSubcoreMesh(core_axis_name="c", subcore_axis_name="s")
_CP = pltpu.CompilerParams(
    needs_layout_passes=False, use_tc_tiling_on_sc=False)

_i32 = jnp.int32
_f32 = jnp.float32


def _wid():
    return lax.axis_index("s") * _NC + lax.axis_index("c")


def _lane():
    return lax.iota(_i32, 16)


def _srl(x, s):
    return lax.shift_right_logical(x, jnp.full((16,), s, _i32))


def _sll(x, s):
    return lax.shift_left(x, jnp.full((16,), s, _i32))


def _splat(x):
    return jnp.zeros((16,), _i32) + x


def _fori(n, body, init):
    return lax.fori_loop(_i32(0), _i32(n), body, init)


def _al(x):
    return pl.multiple_of(x, 8)


def _make_kernels(E):
    M = 2 * E
    CH = M // _NW
    assert CH % _BK == 0 and E % CH == 0
    NB = CH // _BK
    NV = _BK // 16

    @functools.partial(
        pl.kernel,
        out_type=jax.ShapeDtypeStruct((M,), _i32),
        mesh=_MESH, compiler_params=_CP,
        scratch_types=[pltpu.VMEM((_BK,), _i32),
                       pltpu.VMEM((_BK,), _i32),
                       pltpu.VMEM((_BK,), _i32)])
    def k_build(row_h, col_h, key_h, rbuf, cbuf, kbuf):
        base = _wid() * CH

        def bt(t, _):
            b = base + t * _BK
            flip = b >= E
            eb = jnp.where(flip, b - E, b)
            pltpu.sync_copy(row_h.at[pl.ds(_al(eb), _BK)], rbuf)
            pltpu.sync_copy(col_h.at[pl.ds(_al(eb), _BK)], cbuf)

            def bi(i, _2):
                r = rbuf[pl.ds(i * 16, 16)]
                c = cbuf[pl.ds(i * 16, 16)]
                fwd = _sll(r, 16) | c
                rev = _sll(c, 16) | r
                kbuf[pl.ds(i * 16, 16)] = jnp.where(flip, rev, fwd)
                return 0

            _fori(NV, bi, 0)
            pltpu.sync_copy(kbuf, key_h.at[pl.ds(_al(b), _BK)])
            return 0

        _fori(NB, bt, 0)

    def make_hist(shift, bins):
        @functools.partial(
            pl.kernel,
            out_type=jax.ShapeDtypeStruct((_NW * bins,), _i32),
            mesh=_MESH, compiler_params=_CP,
            scratch_types=[pltpu.VMEM((_BK,), _i32),
                           pltpu.VMEM((bins,), _i32)])
        def k_hist(key_h, hist_h, kbuf, htbl):
            w = _wid()
            base = w * CH

            def z(j, _):
                htbl[pl.ds(j * 16, 16)] = jnp.zeros((16,), _i32)
                return 0

            _fori(bins // 16, z, 0)

            def bt(t, _):
                pltpu.sync_copy(key_h.at[pl.ds(_al(base + t * _BK), _BK)], kbuf)

                def bi(i, _2):
                    kk = kbuf[pl.ds(i * 16, 16)]
                    d = _srl(kk, shift) & _i32(bins - 1)
                    rank, lastm = plsc.scan_count(d)
                    plsc.addupdate_scatter(htbl, [d], rank, mask=lastm)
                    return 0

                _fori(NV, bi, 0)
                return 0

            _fori(NB, bt, 0)
            pltpu.sync_copy(htbl, hist_h.at[pl.ds(_al(w * bins), bins)])

        return k_hist

    def make_scat(shift, bins, first):
        ins = 2 if first else 3

        def body(*refs):
            if first:
                key_h, hist_h = refs[0], refs[1]
                id_h = None
                keyo_h, ido_h = refs[2], refs[3]
                rest = refs[4:]
            else:
                key_h, id_h, hist_h = refs[0], refs[1], refs[2]
                keyo_h, ido_h = refs[3], refs[4]
                rest = refs[5:]
            htbl, offtbl, kbuf, ibuf = rest[:4]
            pbufs = list(rest[4:4 + _SS])
            sem = rest[4 + _SS]
            w = _wid()
            base = w * CH
            pltpu.sync_copy(hist_h, htbl)

            def ob(jb, carry):
                tot = jnp.zeros((16,), _i32)
                mine = jnp.zeros((16,), _i32)
                for w2 in range(_NW):
                    hv = htbl[pl.ds(w2 * bins + jb * 16, 16)]
                    tot = tot + hv
                    mine = mine + jnp.where(w2 < w, hv, 0)
                csum = plsc.cumsum(tot)
                offtbl[pl.ds(jb * 16, 16)] = csum - tot + carry + mine
                return carry + jnp.max(csum)

            _fori(bins // 16, ob, _i32(0))

            def bt(t, _):
                b = base + t * _BK
                pltpu.sync_copy(key_h.at[pl.ds(_al(b), _BK)], kbuf)
                if not first:
                    pltpu.sync_copy(id_h.at[pl.ds(_al(b), _BK)], ibuf)

                for k in range(_SS):
                    pbk = pbufs[k]

                    def bi(g, _2, k=k, pbk=pbk):
                        i = k * _SV + g
                        kk = kbuf[pl.ds(i * 16, 16)]
                        d = _srl(kk, shift) & _i32(bins - 1)
                        rank, lastm = plsc.scan_count(d)
                        off = plsc.load_gather(offtbl, [d])
                        pbk[pl.ds(g * 16, 16)] = off + rank - 1
                        plsc.addupdate_scatter(offtbl, [d], rank, mask=lastm)
                        if first:
                            ibuf[pl.ds(i * 16, 16)] = _lane() + (b + i * 16)
                        return 0

                    _fori(_SV, bi, 0)
                hs = []
                for k in range(_SS):
                    hs.append(pltpu.async_copy(
                        kbuf.at[pl.ds(k * _SB, _SB)], keyo_h.at[pbufs[k]], sem))
                    hs.append(pltpu.async_copy(
                        ibuf.at[pl.ds(k * _SB, _SB)], ido_h.at[pbufs[k]], sem))
                for h in hs:
                    h.wait()
                return 0

            _fori(NB, bt, 0)

        return pl.kernel(
            body,
            out_type=(jax.ShapeDtypeStruct((M,), _i32),
                      jax.ShapeDtypeStruct((M,), _i32)),
            mesh=_MESH, compiler_params=_CP,
            scratch_types=[pltpu.VMEM((_NW * bins,), _i32),
                           pltpu.VMEM((bins,), _i32),
                           pltpu.VMEM((_BK,), _i32),
                           pltpu.VMEM((_BK,), _i32)]
                          + [pltpu.VMEM((_SB,), _i32) for _ in range(_SS)]
                          + [pltpu.SemaphoreType.DMA])

    @functools.partial(
        pl.kernel,
        out_type=jax.ShapeDtypeStruct((_NW * 16,), _i32),
        mesh=_MESH, compiler_params=_CP,
        scratch_types=[pltpu.VMEM((_BK + 16,), _i32),
                       pltpu.VMEM((16,), _i32)])
    def k_count(key_h, nseg_h, kext, sbuf):
        w = _wid()
        base = w * CH
        lane = _lane()

        def bt(t, cnt):
            b = base + t * _BK
            pb = jnp.maximum(b - 16, 0)
            pltpu.sync_copy(key_h.at[pl.ds(_al(pb), 16)], kext.at[pl.ds(0, 16)])
            pltpu.sync_copy(key_h.at[pl.ds(_al(b), _BK)], kext.at[pl.ds(16, _BK)])

            def bi(i, c2):
                x = kext[pl.ds(16 + i * 16, 16)]
                xp = kext[pl.ds(15 + i * 16, 16)]
                fl = (x != xp).astype(_i32)
                g0 = jnp.logical_and(b + i * 16 == 0, lane == 0)
                fl = jnp.where(g0, 1, fl)
                return c2 + jnp.sum(fl, dtype=_i32)

            return _fori(NV, bi, cnt)

        cnt = _fori(NB, bt, _i32(0))
        sbuf[...] = _splat(cnt)
        pltpu.sync_copy(sbuf, nseg_h.at[pl.ds(_al(w * 16), 16)])

    @functools.partial(
        pl.kernel,
        out_type=(jax.ShapeDtypeStruct((M + 8, 16), _f32),
                  jax.ShapeDtypeStruct((M + 8,), _i32),
                  jax.ShapeDtypeStruct((M + 8,), _i32),
                  jax.ShapeDtypeStruct((_NW, 16), _f32),
                  jax.ShapeDtypeStruct((_NW * 16,), _i32),
                  jax.ShapeDtypeStruct((_NW * 16,), _i32)),
        mesh=_MESH, compiler_params=_CP,
        scratch_types=[pltpu.VMEM((_BK + 32,), _i32),
                       pltpu.VMEM((_BK,), _i32),
                       pltpu.VMEM((_BK, 16), _f32),
                       pltpu.VMEM((_BK,), _i32),
                       pltpu.VMEM((_BK,), _i32),
                       pltpu.VMEM((_BK,), _i32),
                       pltpu.VMEM((_BK,), _i32),
                       pltpu.VMEM((_BK,), _i32),
                       pltpu.VMEM((_NW * 16,), _i32),
                       pltpu.VMEM((16,), _f32),
                       pltpu.VMEM((16,), _i32)]
                      + [pltpu.VMEM((_SB,), _i32) for _ in range(3 * _SS)]
                      + [pltpu.SemaphoreType.DMA])
    def k_seg(key_h, id_h, attr_h, nseg_h,
              outa_h, us_h, ud_h, tmax_h, tinv_h, cont_h,
              kext, idb, rows, stf, enf, pose, sbuf, dbuf,
              nbuf, v16f, v16i, *rest):
        imbs = list(rest[0:_SS])
        posus = list(rest[_SS:2 * _SS])
        poses = list(rest[2 * _SS:3 * _SS])
        sem = rest[3 * _SS]
        w = _wid()
        base = w * CH
        lane = _lane()
        pltpu.sync_copy(nseg_h, nbuf)
        s0v = jnp.zeros((16,), _i32)
        totv = jnp.zeros((16,), _i32)
        for w2 in range(_NW):
            v = nbuf[pl.ds(w2 * 16, 16)]
            s0v = s0v + jnp.where(w2 < w, v, 0)
            totv = totv + v
        s0 = jnp.max(s0v)
        U = jnp.max(totv)
        neg = jnp.full((16,), -jnp.inf, _f32)

        def bt(t, carry):
            acc, seg = carry
            b = base + t * _BK
            pb = jnp.maximum(b - 16, 0)
            nb2 = jnp.minimum(b + _BK, M - 16)
            pltpu.sync_copy(key_h.at[pl.ds(_al(pb), 16)], kext.at[pl.ds(0, 16)])
            pltpu.sync_copy(key_h.at[pl.ds(_al(b), _BK)], kext.at[pl.ds(16, _BK)])
            pltpu.sync_copy(key_h.at[pl.ds(_al(nb2), 16)],
                            kext.at[pl.ds(16 + _BK, 16)])
            pltpu.sync_copy(id_h.at[pl.ds(_al(b), _BK)], idb)

            segbase = seg
            cc = (seg, _i32(0))
            for k in range(_SS):
                imk, puk, pek = imbs[k], posus[k], poses[k]

                def bi(g, cc2, k=k, imk=imk, puk=puk, pek=pek):
                    segc, fsum = cc2
                    i = k * _SV + g
                    kk = kext[pl.ds(16 + i * 16, 16)]
                    xp = kext[pl.ds(15 + i * 16, 16)]
                    xn = kext[pl.ds(17 + i * 16, 16)]
                    st = (kk != xp).astype(_i32)
                    g0 = jnp.logical_and(b + i * 16 == 0, lane == 0)
                    st = jnp.where(g0, 1, st)
                    en = (kk != xn).astype(_i32)
                    gl = jnp.logical_and(b + i * 16 + 16 == M, lane == 15)
                    en = jnp.where(gl, 1, en)
                    stf[pl.ds(i * 16, 16)] = st
                    enf[pl.ds(i * 16, 16)] = en
                    c = plsc.cumsum(st)
                    loc = segc + c - 1
                    trash = _i32(M) + (lane & 7)
                    puk[pl.ds(g * 16, 16)] = jnp.where(st > 0, s0 + loc, trash)
                    pek[pl.ds(g * 16, 16)] = jnp.where(en > 0, s0 + loc, trash)
                    sbuf[pl.ds(i * 16, 16)] = _srl(kk, 16)
                    dbuf[pl.ds(i * 16, 16)] = kk & _i32(0xFFFF)
                    ii = idb[pl.ds(i * 16, 16)]
                    imk[pl.ds(g * 16, 16)] = ii - jnp.where(
                        ii >= _i32(E), _i32(E), _i32(0))
                    return (segc + jnp.max(c),
                            fsum + jnp.sum(st + en, dtype=_i32))

                cc = _fori(_SV, bi, cc)
            seg, fsum = cc
            hs = [pltpu.async_copy(attr_h.at[imbs[k]],
                                   rows.at[pl.ds(k * _SB, _SB)], sem)
                  for k in range(_SS)]
            for h in hs:
                h.wait()

            def pe(j, a):
                r = rows[j, :]
                ss = plsc.load_gather(stf, [_splat(j)])
                se = plsc.load_gather(enf, [_splat(j)])
                a2 = jnp.maximum(r, jnp.where(ss > 0, neg, a))
                rows[j, :] = jnp.where(se > 0, a2, r)
                return a2

            hw = []
            for k in range(_SS):
                hw.append(pltpu.async_copy(
                    sbuf.at[pl.ds(k * _SB, _SB)], us_h.at[posus[k]], sem))
                hw.append(pltpu.async_copy(
                    dbuf.at[pl.ds(k * _SB, _SB)], ud_h.at[posus[k]], sem))

            def fastb(a):
                pltpu.sync_copy(rows, outa_h.at[pl.ds(s0 + segbase, _BK)])
                return a

            def slowb(a):
                def pv(i, a0):
                    stv = stf[pl.ds(i * 16, 16)]
                    env = enf[pl.ds(i * 16, 16)]
                    allf = jnp.sum(stv + env, dtype=_i32) == 32

                    def slow(a1):
                        return lax.fori_loop(i * 16, i * 16 + 16, pe, a1)

                    return lax.cond(allf, lambda a1: a1, slow, a0)

                a = _fori(NV, pv, a)
                hr = [pltpu.async_copy(rows.at[pl.ds(k * _SB, _SB)],
                                       outa_h.at[poses[k]], sem)
                      for k in range(_SS)]
                for h in hr:
                    h.wait()
                return a

            acc = lax.cond(fsum == 2 * _BK, fastb, slowb, acc)
            for h in hw:
                h.wait()
            return (acc, seg)

        acc, seg = _fori(NB, bt, (jnp.full((16,), -jnp.inf, _f32), _i32(0)))

        v16f[...] = acc
        pltpu.sync_copy(v16f, tmax_h.at[w])
        v16i[...] = _splat(s0 + seg - 1)
        pltpu.sync_copy(v16i, tinv_h.at[pl.ds(_al(w * 16), 16)])
        eb = jnp.minimum(base + CH, M - 16)
        pltpu.sync_copy(key_h.at[pl.ds(_al(base + CH - 16), 16)],
                        kext.at[pl.ds(0, 16)])
        pltpu.sync_copy(key_h.at[pl.ds(_al(eb), 16)], kext.at[pl.ds(16, 16)])
        x1 = kext[pl.ds(0, 16)]
        x2 = kext[pl.ds(16, 16)]
        e_last = jnp.sum(jnp.where(lane == 15, x1, 0), dtype=_i32)
        e_next = jnp.sum(jnp.where(lane == 0, x2, 0), dtype=_i32)
        cont = jnp.logical_and(w < _NW - 1, e_last == e_next)
        v16i[...] = _splat(cont.astype(_i32))
        pltpu.sync_copy(v16i, cont_h.at[pl.ds(_al(w * 16), 16)])

        lo = jnp.maximum(U, w * CH)
        hi = (w + 1) * CH

        def zr(j, _):
            rows[j, :] = jnp.zeros((16,), _f32)
            return 0

        _fori(_BK, zr, 0)

        def fv(i, _):
            sbuf[pl.ds(i * 16, 16)] = jnp.full((16,), -1, _i32)
            dbuf[pl.ds(i * 16, 16)] = jnp.full((16,), _N - 1, _i32)
            return 0

        _fori(NV, fv, 0)
        nblk = jnp.maximum(0, (hi - lo + _BK - 1) // _BK)

        def fb(q, _):
            st = lo + q * _BK

            def fi(i, _2):
                v = jnp.minimum(st + i * 16 + lane, hi - 1)
                pose[pl.ds(i * 16, 16)] = v
                return 0

            _fori(NV, fi, 0)
            c1 = pltpu.async_copy(rows, outa_h.at[pose], sem)
            c2 = pltpu.async_copy(sbuf, us_h.at[pose], sem)
            c3 = pltpu.async_copy(dbuf, ud_h.at[pose], sem)
            c1.wait()
            c2.wait()
            c3.wait()
            return 0

        _fori(nblk, fb, 0)

    hists = [make_hist(s, 1 << b) for s, b in zip(_SHIFTS, _BITS)]
    scats = [make_scat(s, 1 << b, i == 0)
             for i, (s, b) in enumerate(zip(_SHIFTS, _BITS))]
    return k_build, hists, scats, k_count, k_seg


def kernel(edge_index, edge_attr):
    E = edge_index.shape[1]
    M = 2 * E
    k_build, hists, scats, k_count, k_seg = _make_kernels(E)

    ei = edge_index.astype(jnp.int32)
    row, col = ei[0], ei[1]
    attr = edge_attr.astype(jnp.float32)

    key = k_build(row, col)
    h = hists[0](key)
    key, ids = scats[0](key, h)
    h = hists[1](key)
    key, ids = scats[1](key, ids, h)
    h = hists[2](key)
    key, ids = scats[2](key, ids, h)
    nseg = k_count(key)
    outa, us, ud, tmax, tinv, cont = k_seg(key, ids, attr, nseg)

    tinv0 = tinv.reshape(_NW, 16)[:, 0]
    cont0 = cont.reshape(_NW, 16)[:, 0]
    prow = jnp.where(cont0 > 0, tinv0, jnp.int32(M))
    outa = outa.at[prow].max(tmax)

    out = outa[:M]
    src = us[:M].astype(jnp.int64)
    dst = ud[:M].astype(jnp.int64)
    new_edge_index = jnp.stack([src, dst])
    return new_edge_index, out
